# Initial kernel scaffold; baseline (speedup 1.0000x reference)
import functools

import jax
import jax.numpy as jnp
from jax import lax
from jax.experimental import pallas as pl
from jax.experimental.pallas import tpu as pltpu
from jax.experimental.pallas import tpu_sc as plsc

DIM = 64
N_EMBED = 8192
N_TOK = 16 * 1024
BETA = 0.25

TM = 1024
TN = 2048
N_I = N_TOK // TM
N_J = N_EMBED // TN


def _proj_body(embed_ref, w_ref, b_ref, out_ref):
    out_ref[...] = (
        lax.dot_general(embed_ref[...], w_ref[...],
                        (((1,), (1,)), ((), ())))
        + b_ref[...][None, :]
    )


def _project_codebook(embed, proj_w, proj_b):
    return pl.pallas_call(
        _proj_body,
        out_shape=jax.ShapeDtypeStruct((N_EMBED, DIM), jnp.float32),
    )(embed, proj_w, proj_b)


def _dist_body(x_ref, c_ref, d_ref, idx_ref, mind_ref, dsum_ref,
               run_min, run_idx, acc):
    j = pl.program_id(1)
    x = x_ref[...]
    c = c_ref[...]
    sumx = jnp.sum(x * x, axis=1, keepdims=True)
    sumc = jnp.sum(c * c, axis=1, keepdims=True)
    dot = lax.dot_general(x, c, (((1,), (1,)), ((), ())))
    d = (sumx + sumc.T) - 2.0 * dot
    d_ref[...] = d

    tile_min = jnp.min(d, axis=1, keepdims=True)
    col = jax.lax.broadcasted_iota(jnp.int32, d.shape, 1)
    big = jnp.int32(N_EMBED)
    tile_idx = jnp.min(jnp.where(d == tile_min, col, big),
                       axis=1, keepdims=True) + j * TN

    @pl.when(j == 0)
    def _init():
        run_min[...] = tile_min
        run_idx[...] = tile_idx

    @pl.when(j > 0)
    def _update():
        better = tile_min < run_min[...]
        run_min[...] = jnp.where(better, tile_min, run_min[...])
        run_idx[...] = jnp.where(better, tile_idx, run_idx[...])

    @pl.when(j == N_J - 1)
    def _final():
        idx_ref[...] = run_idx[...]
        mind_ref[...] = run_min[...]
        tile_sum = jnp.sum(run_min[...])

        @pl.when(pl.program_id(0) == 0)
        def _first():
            acc[0] = tile_sum

        @pl.when(pl.program_id(0) > 0)
        def _rest():
            acc[0] = acc[0] + tile_sum

        dsum_ref[0, 0] = acc[0]


def _distance_argmin(flat, qcb):
    return pl.pallas_call(
        _dist_body,
        grid=(N_I, N_J),
        in_specs=[
            pl.BlockSpec((TM, DIM), lambda i, j: (i, 0)),
            pl.BlockSpec((TN, DIM), lambda i, j: (j, 0)),
        ],
        out_specs=[
            pl.BlockSpec((TM, TN), lambda i, j: (i, j)),
            pl.BlockSpec((TM, 1), lambda i, j: (i, 0)),
            pl.BlockSpec((TM, 1), lambda i, j: (i, 0)),
            pl.BlockSpec((1, 1), lambda i, j: (0, 0)),
        ],
        out_shape=[
            jax.ShapeDtypeStruct((N_TOK, N_EMBED), jnp.float32),
            jax.ShapeDtypeStruct((N_TOK, 1), jnp.int32),
            jax.ShapeDtypeStruct((N_TOK, 1), jnp.float32),
            jax.ShapeDtypeStruct((1, 1), jnp.float32),
        ],
        scratch_shapes=[
            pltpu.VMEM((TM, 1), jnp.float32),
            pltpu.VMEM((TM, 1), jnp.int32),
            pltpu.SMEM((1,), jnp.float32),
        ],
    )(flat, qcb)


def _make_sc_gather():
    info = plsc.get_sparse_core_info()
    nw = info.num_cores * info.num_subcores
    b_per_w = N_TOK // nw
    mesh = plsc.VectorSubcoreMesh(core_axis_name="c", subcore_axis_name="s")

    @functools.partial(
        pl.kernel, mesh=mesh,
        out_type=jax.ShapeDtypeStruct((N_TOK, DIM), jnp.float32),
        scratch_types=[
            pltpu.VMEM((b_per_w,), jnp.int32),
            pltpu.VMEM((b_per_w, DIM), jnp.float32),
            pltpu.SemaphoreType.DMA,
        ],
    )
    def gather(table_hbm, idx_hbm, out_hbm, idx_v, rows_v, sem):
        wid = lax.axis_index("s") * info.num_cores + lax.axis_index("c")
        base = wid * b_per_w
        pltpu.sync_copy(idx_hbm.at[pl.ds(base, b_per_w)], idx_v)
        pltpu.async_copy(table_hbm.at[idx_v], rows_v, sem).wait()
        pltpu.sync_copy(rows_v, out_hbm.at[pl.ds(base, b_per_w)])

    return gather


def kernel(input, is_look_back, embed, proj_w, proj_b):
    flat = input.reshape(-1, DIM)
    qcb = _project_codebook(embed, proj_w, proj_b)
    d, idx, _mind, dsum = _distance_argmin(flat, qcb)
    idx_flat = idx.reshape(-1)
    zq = _make_sc_gather()(qcb, idx_flat)
    z_quantize = zq.reshape(input.shape)
    diff = (1.0 + BETA) * dsum[0, 0] / jnp.float32(N_TOK * DIM)
    embed_ind = idx.reshape(input.shape[:-1])
    return (z_quantize, diff, embed_ind, d)


# speedup vs baseline: 2.3213x; 2.3213x over previous
import functools

import jax
import jax.numpy as jnp
from jax import lax
from jax.experimental import pallas as pl
from jax.experimental.pallas import tpu as pltpu
from jax.experimental.pallas import tpu_sc as plsc

DIM = 64
N_EMBED = 8192
N_TOK = 16 * 1024
BETA = 0.25

TM = 1024
TN = 2048
N_I = N_TOK // TM
N_J = N_EMBED // TN


def _proj_body(embed_ref, w_ref, b_ref, out_ref):
    out_ref[...] = (
        lax.dot_general(embed_ref[...], w_ref[...],
                        (((1,), (1,)), ((), ())))
        + b_ref[...][None, :]
    )


def _project_codebook(embed, proj_w, proj_b):
    return pl.pallas_call(
        _proj_body,
        out_shape=jax.ShapeDtypeStruct((N_EMBED, DIM), jnp.float32),
    )(embed, proj_w, proj_b)


def _dist_body(x_ref, c_ref, d_ref, idx_ref, mind_ref, dsum_ref,
               run_min, run_idx, acc):
    i = pl.program_id(0)
    j = pl.program_id(1)
    x = x_ref[...]
    c = c_ref[...]
    sumx = jnp.sum(x * x, axis=1, keepdims=True)
    sumc = jnp.sum(c * c, axis=1, keepdims=True)
    dot = lax.dot_general(x, c, (((1,), (1,)), ((), ())))
    d = (sumx + sumc.T) - 2.0 * dot
    d_ref[...] = d

    tile_min = jnp.min(d, axis=1, keepdims=True)
    col = jax.lax.broadcasted_iota(jnp.int32, d.shape, 1)
    big = jnp.int32(N_EMBED)
    tile_idx = jnp.min(jnp.where(d == tile_min, col, big),
                       axis=1, keepdims=True) + j * TN

    @pl.when(j == 0)
    def _init():
        run_min[...] = tile_min
        run_idx[...] = tile_idx

    @pl.when(j > 0)
    def _update():
        better = tile_min < run_min[...]
        run_min[...] = jnp.where(better, tile_min, run_min[...])
        run_idx[...] = jnp.where(better, tile_idx, run_idx[...])

    @pl.when(j == N_J - 1)
    def _final():
        idx_ref[...] = run_idx[...]
        mind_ref[...] = run_min[...]
        tile_sum = jnp.sum(run_min[...])

        @pl.when(i == 0)
        def _first():
            acc[0] = tile_sum

        @pl.when(i > 0)
        def _rest():
            acc[0] = acc[0] + tile_sum

        dsum_ref[0, 0] = acc[0]


def _distance_argmin(flat, qcb):
    return pl.pallas_call(
        _dist_body,
        grid=(N_I, N_J),
        in_specs=[
            pl.BlockSpec((TM, DIM), lambda i, j: (i, 0)),
            pl.BlockSpec((TN, DIM), lambda i, j: (j, 0)),
        ],
        out_specs=[
            pl.BlockSpec((TM, TN), lambda i, j: (i, j)),
            pl.BlockSpec((TM, 1), lambda i, j: (i, 0)),
            pl.BlockSpec((TM, 1), lambda i, j: (i, 0)),
            pl.BlockSpec((1, 1), lambda i, j: (0, 0),
                         memory_space=pltpu.SMEM),
        ],
        out_shape=[
            jax.ShapeDtypeStruct((N_TOK, N_EMBED), jnp.float32),
            jax.ShapeDtypeStruct((N_TOK, 1), jnp.int32),
            jax.ShapeDtypeStruct((N_TOK, 1), jnp.float32),
            jax.ShapeDtypeStruct((1, 1), jnp.float32),
        ],
        scratch_shapes=[
            pltpu.VMEM((TM, 1), jnp.float32),
            pltpu.VMEM((TM, 1), jnp.int32),
            pltpu.SMEM((1,), jnp.float32),
        ],
    )(flat, qcb)


def _make_sc_gather():
    info = plsc.get_sparse_core_info()
    nw = info.num_cores * info.num_subcores
    b_per_w = N_TOK // nw
    mesh = plsc.VectorSubcoreMesh(core_axis_name="c", subcore_axis_name="s")

    @functools.partial(
        pl.kernel, mesh=mesh,
        out_type=jax.ShapeDtypeStruct((N_TOK, DIM), jnp.float32),
        compiler_params=pltpu.CompilerParams(use_tc_tiling_on_sc=False),
        scratch_types=[
            pltpu.VMEM((b_per_w,), jnp.int32),
            pltpu.VMEM((b_per_w, DIM), jnp.float32),
            pltpu.SemaphoreType.DMA,
        ],
    )
    def gather(table_hbm, idx_hbm, out_hbm, idx_v, rows_v, sem):
        wid = lax.axis_index("s") * info.num_cores + lax.axis_index("c")
        base = wid * b_per_w
        pltpu.sync_copy(idx_hbm.at[pl.ds(base, b_per_w)], idx_v)
        pltpu.async_copy(table_hbm.at[idx_v], rows_v, sem).wait()
        pltpu.sync_copy(rows_v, out_hbm.at[pl.ds(base, b_per_w)])

    return gather


def kernel(input, is_look_back, embed, proj_w, proj_b):
    flat = input.reshape(-1, DIM)
    qcb = _project_codebook(embed, proj_w, proj_b)
    d, idx, _mind, dsum = _distance_argmin(flat, qcb)
    idx_flat = idx.reshape(-1)
    zq = _make_sc_gather()(qcb, idx_flat)
    z_quantize = zq.reshape(input.shape)
    diff = (1.0 + BETA) * dsum[0, 0] / jnp.float32(N_TOK * DIM)
    embed_ind = idx.reshape(input.shape[:-1])
    return (z_quantize, diff, embed_ind, d)


# speedup vs baseline: 3.3068x; 1.4246x over previous
import functools

import jax
import jax.numpy as jnp
from jax import lax
from jax.experimental import pallas as pl
from jax.experimental.pallas import tpu as pltpu
from jax.experimental.pallas import tpu_sc as plsc

DIM = 64
N_EMBED = 8192
N_TOK = 16 * 1024
BETA = 0.25

TM = 512
N_I = N_TOK // TM


def _proj_body(embed_ref, w_ref, b_ref, qcb_ref, cn_ref):
    emb = embed_ref[...]
    w = w_ref[...]
    b = b_ref[...]
    qcb = lax.dot_general(emb, w, (((1,), (1,)), ((), ()))) + b[None, :]
    qcb_ref[...] = qcb
    sumc = jnp.sum(qcb * qcb, axis=1, keepdims=True)
    cn_ref[...] = sumc.T


def _project_codebook(embed, proj_w, proj_b):
    return pl.pallas_call(
        _proj_body,
        out_shape=[
            jax.ShapeDtypeStruct((N_EMBED, DIM), jnp.float32),
            jax.ShapeDtypeStruct((1, N_EMBED), jnp.float32),
        ],
    )(embed, proj_w, proj_b)


def _dist_body(x_ref, ct_ref, cn_ref, d_ref, idx_ref, dsum_ref, acc):
    i = pl.program_id(0)
    x = x_ref[...]
    sumx = jnp.sum(x * x, axis=1, keepdims=True)
    dot = lax.dot_general(x, ct_ref[...],
                          (((1,), (1,)), ((), ())))
    d = (sumx + cn_ref[...]) - 2.0 * dot
    d_ref[...] = d

    m = jnp.min(d, axis=1, keepdims=True)
    col = jax.lax.broadcasted_iota(jnp.int32, d.shape, 1)
    idx_ref[...] = jnp.min(jnp.where(d == m, col, jnp.int32(N_EMBED)),
                           axis=1, keepdims=True)
    tile_sum = jnp.sum(m)

    @pl.when(i == 0)
    def _first():
        acc[0] = tile_sum

    @pl.when(i > 0)
    def _rest():
        acc[0] = acc[0] + tile_sum

    dsum_ref[0, 0] = acc[0]


def _distance_argmin(flat, qcbt, cn):
    return pl.pallas_call(
        _dist_body,
        grid=(N_I,),
        in_specs=[
            pl.BlockSpec((TM, DIM), lambda i: (i, 0)),
            pl.BlockSpec((N_EMBED, DIM), lambda i: (0, 0)),
            pl.BlockSpec((1, N_EMBED), lambda i: (0, 0)),
        ],
        out_specs=[
            pl.BlockSpec((TM, N_EMBED), lambda i: (i, 0)),
            pl.BlockSpec((TM, 1), lambda i: (i, 0)),
            pl.BlockSpec((1, 1), lambda i: (0, 0),
                         memory_space=pltpu.SMEM),
        ],
        out_shape=[
            jax.ShapeDtypeStruct((N_TOK, N_EMBED), jnp.float32),
            jax.ShapeDtypeStruct((N_TOK, 1), jnp.int32),
            jax.ShapeDtypeStruct((1, 1), jnp.float32),
        ],
        scratch_shapes=[
            pltpu.SMEM((1,), jnp.float32),
        ],
    )(flat, qcbt, cn)


def _make_sc_gather():
    info = plsc.get_sparse_core_info()
    nw = info.num_cores * info.num_subcores
    b_per_w = N_TOK // nw
    mesh = plsc.VectorSubcoreMesh(core_axis_name="c", subcore_axis_name="s")

    @functools.partial(
        pl.kernel, mesh=mesh,
        out_type=jax.ShapeDtypeStruct((N_TOK, DIM), jnp.float32),
        compiler_params=pltpu.CompilerParams(use_tc_tiling_on_sc=False),
        scratch_types=[
            pltpu.VMEM((b_per_w,), jnp.int32),
            pltpu.VMEM((b_per_w, DIM), jnp.float32),
            pltpu.SemaphoreType.DMA,
        ],
    )
    def gather(table_hbm, idx_hbm, out_hbm, idx_v, rows_v, sem):
        wid = lax.axis_index("s") * info.num_cores + lax.axis_index("c")
        base = wid * b_per_w
        pltpu.sync_copy(idx_hbm.at[pl.ds(base, b_per_w)], idx_v)
        pltpu.async_copy(table_hbm.at[idx_v], rows_v, sem).wait()
        pltpu.sync_copy(rows_v, out_hbm.at[pl.ds(base, b_per_w)])

    return gather


def kernel(input, is_look_back, embed, proj_w, proj_b):
    flat = input.reshape(-1, DIM)
    qcb, cn = _project_codebook(embed, proj_w, proj_b)
    d, idx, dsum = _distance_argmin(flat, qcb, cn)
    idx_flat = idx.reshape(-1)
    zq = _make_sc_gather()(qcb, idx_flat)
    z_quantize = zq.reshape(input.shape)
    diff = (1.0 + BETA) * dsum[0, 0] / jnp.float32(N_TOK * DIM)
    embed_ind = idx.reshape(input.shape[:-1])
    return (z_quantize, diff, embed_ind, d)
